# baseline (speedup 1.0000x reference)
import functools

import numpy as np

import jax
import jax.numpy as jnp
from jax import lax
from jax.experimental import pallas as pl
from jax.experimental.pallas import tpu as pltpu
from jax.experimental.pallas import tpu_sc as plsc

B = 16384
V = 100000
N_GENRES = 20
OV_DIM = 384
COLL_DIM = 32
MOVIE_DIM = 64
RANK = 64

NC = 2
NS = 16
NW = NC * NS
B_PER_W = B // NW
CHUNK = 128
NCHUNK = B_PER_W // CHUNK

CDIM = 128
RD_COL = MOVIE_DIM + N_GENRES

TWO32 = 4294967296.0
DAY_NS_INV = 1.0 / 86_400e9


def _sc_ov_body(idx_hbm, ov_hbm, ov_out, idx_v, ov0, ov1, sem0, sem1):
    i32 = jnp.int32
    wid = (lax.axis_index("s") * NC + lax.axis_index("c")).astype(i32)
    base = wid * i32(B_PER_W)
    pltpu.sync_copy(idx_hbm.at[wid], idx_v)
    bufs = (ov0, ov1)
    sems = (sem0, sem1)
    cps = [None, None]
    for c in range(NCHUNK):
        cps[c % 2] = pltpu.async_copy(
            ov_hbm.at[idx_v.at[i32(c)]], bufs[c % 2], sems[c % 2])
        if c > 0:
            cps[(c - 1) % 2].wait()
            pltpu.sync_copy(bufs[(c - 1) % 2],
                            ov_out.at[pl.ds(base + i32((c - 1) * CHUNK), CHUNK)])
    cps[(NCHUNK - 1) % 2].wait()
    pltpu.sync_copy(bufs[(NCHUNK - 1) % 2],
                    ov_out.at[pl.ds(base + i32((NCHUNK - 1) * CHUNK), CHUNK)])


def _sc_ov_gather(idx3, ov_emb):
    mesh = plsc.VectorSubcoreMesh(core_axis_name="c", subcore_axis_name="s")
    f32, i32 = jnp.float32, jnp.int32
    run = pl.kernel(
        _sc_ov_body,
        out_type=[jax.ShapeDtypeStruct((B, OV_DIM), f32)],
        mesh=mesh,
        scratch_types=[
            pltpu.VMEM((NCHUNK, CHUNK), i32),
            pltpu.VMEM((CHUNK, OV_DIM), f32),
            pltpu.VMEM((CHUNK, OV_DIM), f32),
            pltpu.SemaphoreType.DMA,
            pltpu.SemaphoreType.DMA,
        ],
    )
    return run(idx3, ov_emb)


def _sc_small_body(idx_hbm, comb_hbm, coll_hbm, wcoll_hbm,
                   comb_out, cw_out,
                   idx_v, cbv, cidv, cwv, sem_a, sem_b):
    i32 = jnp.int32
    wid = (lax.axis_index("s") * NC + lax.axis_index("c")).astype(i32)
    base = wid * i32(B_PER_W)
    pltpu.sync_copy(idx_hbm.at[wid], idx_v)
    for c in range(NCHUNK):
        ids = idx_v.at[i32(c)]
        off = base + i32(c * CHUNK)
        cp_id = pltpu.async_copy(coll_hbm.at[ids], cidv, sem_b)
        cp_cb = pltpu.async_copy(comb_hbm.at[ids], cbv, sem_a)
        cp_id.wait()
        cp_cw = pltpu.async_copy(wcoll_hbm.at[cidv], cwv, sem_b)
        cp_cb.wait()
        pltpu.sync_copy(cbv, comb_out.at[pl.ds(off, CHUNK)])
        cp_cw.wait()
        pltpu.sync_copy(cwv, cw_out.at[pl.ds(off, CHUNK)])


def _sc_small_gather(idx3, comb, coll32, wcoll_p):
    mesh = plsc.VectorSubcoreMesh(core_axis_name="c", subcore_axis_name="s")
    f32, i32 = jnp.float32, jnp.int32
    run = pl.kernel(
        _sc_small_body,
        out_type=[
            jax.ShapeDtypeStruct((B, CDIM), f32),
            jax.ShapeDtypeStruct((B, CDIM), f32),
        ],
        mesh=mesh,
        scratch_types=[
            pltpu.VMEM((NCHUNK, CHUNK), i32),
            pltpu.VMEM((CHUNK, CDIM), f32),
            pltpu.VMEM((CHUNK,), i32),
            pltpu.VMEM((CHUNK, CDIM), f32),
            pltpu.SemaphoreType.DMA,
            pltpu.SemaphoreType.DMA,
        ],
    )
    return run(idx3, comb, coll32, wcoll_p)


BMB = 2000


def _tc_build_body(mv_ref, gv_ref, rd_ref, out_ref):
    out_ref[...] = jnp.concatenate(
        [mv_ref[...], gv_ref[...], rd_ref[...],
         jnp.zeros((BMB, CDIM - RD_COL - 2), jnp.float32)], axis=1)


def _tc_build(movie_emb, genres, rd2f):
    zero = np.int32(0)

    def row_block(d):
        return pl.BlockSpec((BMB, d), lambda i: (i, zero))

    return pl.pallas_call(
        _tc_build_body,
        grid=(V // BMB,),
        in_specs=[row_block(MOVIE_DIM), row_block(N_GENRES), row_block(2)],
        out_specs=pl.BlockSpec((BMB, CDIM), lambda i: (i, np.int32(0))),
        out_shape=jax.ShapeDtypeStruct((V, CDIM), jnp.float32),
    )(movie_emb, genres, rd2f)


def _split_f32(lo, hi):
    f32 = jnp.float32
    lo_f = lo.astype(f32) + jnp.where(lo < 0, f32(TWO32), f32(0.0))
    return hi.astype(f32) * f32(TWO32) + lo_f


def _tc_mlp_body(ov_ref, cb_ref, cw_ref, ts_ref,
                 w1cb_ref, w1cw_ref, w1o_ref, wts_ref, b1_ref,
                 w2_ref, b2_ref, out_ref):
    comb = cb_ref[...]
    rd_lo = lax.bitcast_convert_type(comb[:, RD_COL:RD_COL + 1], jnp.int32)
    rd_hi = lax.bitcast_convert_type(comb[:, RD_COL + 1:RD_COL + 2], jnp.int32)
    ts = ts_ref[...]
    f32 = jnp.float32
    diff = _split_f32(ts[:, 0:1], ts[:, 1:2]) - _split_f32(rd_lo, rd_hi)
    days = jnp.maximum(jnp.floor(diff * f32(DAY_NS_INV)), f32(1.0))
    x_ts = (jnp.minimum(jnp.log(days), f32(10.0)) - f32(5.0)) / f32(5.0)
    dot = functools.partial(jnp.dot, preferred_element_type=jnp.float32)
    h = (dot(comb, w1cb_ref[...])
         + dot(cw_ref[...], w1cw_ref[...])
         + dot(ov_ref[...], w1o_ref[...])
         + x_ts * wts_ref[...]
         + b1_ref[...])
    h = jnp.maximum(h, 0.0)
    out_ref[...] = dot(h, w2_ref[...]) + b2_ref[...]


def _tc_mlp(ov_g, comb_g, cw_g, ts2, w1cb, w1cw, w1o, wts, b1r, w2, b2r):
    BM = 1024
    zero = np.int32(0)

    def row_block(d):
        return pl.BlockSpec((BM, d), lambda i: (i, zero))

    def full_block(shape):
        return pl.BlockSpec(shape, lambda i: (zero, zero))

    return pl.pallas_call(
        _tc_mlp_body,
        grid=(B // BM,),
        in_specs=[
            row_block(OV_DIM),
            row_block(CDIM),
            row_block(CDIM),
            row_block(2),
            full_block((CDIM, 64)),
            full_block((CDIM, 64)),
            full_block((OV_DIM, 64)),
            full_block((1, 64)),
            full_block((1, 64)),
            full_block((64, RANK)),
            full_block((1, RANK)),
        ],
        out_specs=pl.BlockSpec((BM, RANK), lambda i: (i, np.int32(0))),
        out_shape=jax.ShapeDtypeStruct((B, RANK), jnp.float32),
    )(ov_g, comb_g, cw_g, ts2, w1cb, w1cw, w1o, wts, b1r, w2, b2r)


def kernel(x, ts, movie_emb, genres, collection, ov_emb, release_date,
           W_coll, W1, b1, W2, b2):
    f32 = jnp.float32
    idx3 = x.astype(jnp.int32).reshape(NW, NCHUNK, CHUNK)
    coll32 = collection.astype(jnp.int32)
    ts2 = lax.bitcast_convert_type(ts, jnp.int32)
    rd2f = lax.bitcast_convert_type(
        lax.bitcast_convert_type(release_date, jnp.int32), f32)

    wcoll_p = jnp.concatenate(
        [W_coll, jnp.zeros((W_coll.shape[0], CDIM - COLL_DIM), f32)], axis=1)

    (ov_g,) = _sc_ov_gather(idx3, ov_emb)
    comb = _tc_build(movie_emb, genres, rd2f)
    comb_g, cw_g = _sc_small_gather(idx3, comb, coll32, wcoll_p)

    w1cb = jnp.concatenate([W1[:RD_COL], jnp.zeros((CDIM - RD_COL, 64), f32)])
    w1cw = jnp.concatenate(
        [W1[RD_COL:RD_COL + COLL_DIM], jnp.zeros((CDIM - COLL_DIM, 64), f32)])
    w1o = W1[RD_COL + COLL_DIM:RD_COL + COLL_DIM + OV_DIM]
    wts = W1[RD_COL + COLL_DIM + OV_DIM:]
    return _tc_mlp(ov_g, comb_g, cw_g, ts2, w1cb, w1cw, w1o, wts,
                   b1.reshape(1, 64), W2, b2.reshape(1, RANK))


# speedup vs baseline: 1.4048x; 1.4048x over previous
import functools

import numpy as np

import jax
import jax.numpy as jnp
from jax import lax
from jax.experimental import pallas as pl
from jax.experimental.pallas import tpu as pltpu
from jax.experimental.pallas import tpu_sc as plsc

B = 16384
V = 100000
N_GENRES = 20
OV_DIM = 384
COLL_DIM = 32
MOVIE_DIM = 64
RANK = 64

NC = 2
NS = 16
NW = NC * NS
B_PER_W = B // NW
CHUNK = 128
NCHUNK = B_PER_W // CHUNK

CDIM = 128
RD_COL = MOVIE_DIM + N_GENRES

TWO32 = 4294967296.0
DAY_NS_INV = 1.0 / 86_400e9


def _sc_ov_body(idx_hbm, ov_hbm, ov_out, idx_v, ov0, ov1, sem0, sem1):
    i32 = jnp.int32
    wid = (lax.axis_index("s") * NC + lax.axis_index("c")).astype(i32)
    base = wid * i32(B_PER_W)
    pltpu.sync_copy(idx_hbm.at[wid], idx_v)
    bufs = (ov0, ov1)
    sems = (sem0, sem1)
    cps = [None, None]
    for c in range(NCHUNK):
        cps[c % 2] = pltpu.async_copy(
            ov_hbm.at[idx_v.at[i32(c)]], bufs[c % 2], sems[c % 2])
        if c > 0:
            cps[(c - 1) % 2].wait()
            pltpu.sync_copy(bufs[(c - 1) % 2],
                            ov_out.at[pl.ds(base + i32((c - 1) * CHUNK), CHUNK)])
    cps[(NCHUNK - 1) % 2].wait()
    pltpu.sync_copy(bufs[(NCHUNK - 1) % 2],
                    ov_out.at[pl.ds(base + i32((NCHUNK - 1) * CHUNK), CHUNK)])


def _sc_ov_gather(idx3, ov_emb):
    mesh = plsc.VectorSubcoreMesh(core_axis_name="c", subcore_axis_name="s")
    f32, i32 = jnp.float32, jnp.int32
    run = pl.kernel(
        _sc_ov_body,
        out_type=[jax.ShapeDtypeStruct((B, OV_DIM), f32)],
        mesh=mesh,
        scratch_types=[
            pltpu.VMEM((NCHUNK, CHUNK), i32),
            pltpu.VMEM((CHUNK, OV_DIM), f32),
            pltpu.VMEM((CHUNK, OV_DIM), f32),
            pltpu.SemaphoreType.DMA,
            pltpu.SemaphoreType.DMA,
        ],
    )
    return run(idx3, ov_emb)


def _sc_small_body(idx_hbm, comb_hbm, coll_hbm, wcoll_hbm,
                   comb_out, cw_out,
                   idx_v, cbv, cidv, cwv, sem_a, sem_b):
    i32 = jnp.int32
    wid = (lax.axis_index("s") * NC + lax.axis_index("c")).astype(i32)
    base = wid * i32(B_PER_W)
    pltpu.sync_copy(idx_hbm.at[wid], idx_v)
    for c in range(NCHUNK):
        ids = idx_v.at[i32(c)]
        off = base + i32(c * CHUNK)
        cp_id = pltpu.async_copy(coll_hbm.at[ids], cidv, sem_b)
        cp_cb = pltpu.async_copy(comb_hbm.at[ids], cbv, sem_a)
        cp_id.wait()
        cp_cw = pltpu.async_copy(wcoll_hbm.at[cidv], cwv, sem_b)
        cp_cb.wait()
        pltpu.sync_copy(cbv, comb_out.at[pl.ds(off, CHUNK)])
        cp_cw.wait()
        pltpu.sync_copy(cwv, cw_out.at[pl.ds(off, CHUNK)])


def _sc_small_gather(idx3, comb, coll32, wcoll_p):
    mesh = plsc.VectorSubcoreMesh(core_axis_name="c", subcore_axis_name="s")
    f32, i32 = jnp.float32, jnp.int32
    run = pl.kernel(
        _sc_small_body,
        out_type=[
            jax.ShapeDtypeStruct((B, CDIM), f32),
            jax.ShapeDtypeStruct((B, CDIM), f32),
        ],
        mesh=mesh,
        scratch_types=[
            pltpu.VMEM((NCHUNK, CHUNK), i32),
            pltpu.VMEM((CHUNK, CDIM), f32),
            pltpu.VMEM((CHUNK,), i32),
            pltpu.VMEM((CHUNK, CDIM), f32),
            pltpu.SemaphoreType.DMA,
            pltpu.SemaphoreType.DMA,
        ],
    )
    return run(idx3, comb, coll32, wcoll_p)


def _split_f32(lo, hi):
    f32 = jnp.float32
    lo_f = lo.astype(f32) + jnp.where(lo < 0, f32(TWO32), f32(0.0))
    return hi.astype(f32) * f32(TWO32) + lo_f


def _tc_mlp_body(ov_ref, cb_ref, cw_ref, ts_ref,
                 w1cb_ref, w1cw_ref, w1o_ref, wts_ref, b1_ref,
                 w2_ref, b2_ref, out_ref):
    comb = cb_ref[...]
    rd_lo = lax.bitcast_convert_type(comb[:, RD_COL:RD_COL + 1], jnp.int32)
    rd_hi = lax.bitcast_convert_type(comb[:, RD_COL + 1:RD_COL + 2], jnp.int32)
    ts = ts_ref[...]
    f32 = jnp.float32
    diff = _split_f32(ts[:, 0:1], ts[:, 1:2]) - _split_f32(rd_lo, rd_hi)
    days = jnp.maximum(jnp.floor(diff * f32(DAY_NS_INV)), f32(1.0))
    x_ts = (jnp.minimum(jnp.log(days), f32(10.0)) - f32(5.0)) / f32(5.0)
    dot = functools.partial(jnp.dot, preferred_element_type=jnp.float32)
    h = (dot(comb, w1cb_ref[...])
         + dot(cw_ref[...], w1cw_ref[...])
         + dot(ov_ref[...], w1o_ref[...])
         + x_ts * wts_ref[...]
         + b1_ref[...])
    h = jnp.maximum(h, 0.0)
    out_ref[...] = dot(h, w2_ref[...]) + b2_ref[...]


def _tc_mlp(ov_g, comb_g, cw_g, ts2, w1cb, w1cw, w1o, wts, b1r, w2, b2r):
    BM = 1024
    zero = np.int32(0)

    def row_block(d):
        return pl.BlockSpec((BM, d), lambda i: (i, zero))

    def full_block(shape):
        return pl.BlockSpec(shape, lambda i: (zero, zero))

    return pl.pallas_call(
        _tc_mlp_body,
        grid=(B // BM,),
        in_specs=[
            row_block(OV_DIM),
            row_block(CDIM),
            row_block(CDIM),
            row_block(2),
            full_block((CDIM, 64)),
            full_block((CDIM, 64)),
            full_block((OV_DIM, 64)),
            full_block((1, 64)),
            full_block((1, 64)),
            full_block((64, RANK)),
            full_block((1, RANK)),
        ],
        out_specs=pl.BlockSpec((BM, RANK), lambda i: (i, np.int32(0))),
        out_shape=jax.ShapeDtypeStruct((B, RANK), jnp.float32),
    )(ov_g, comb_g, cw_g, ts2, w1cb, w1cw, w1o, wts, b1r, w2, b2r)


def kernel(x, ts, movie_emb, genres, collection, ov_emb, release_date,
           W_coll, W1, b1, W2, b2):
    f32 = jnp.float32
    idx3 = x.astype(jnp.int32).reshape(NW, NCHUNK, CHUNK)
    coll32 = collection.astype(jnp.int32)
    ts2 = lax.bitcast_convert_type(ts, jnp.int32)
    rd2f = lax.bitcast_convert_type(
        lax.bitcast_convert_type(release_date, jnp.int32), f32)

    wcoll_p = jnp.concatenate(
        [W_coll, jnp.zeros((W_coll.shape[0], CDIM - COLL_DIM), f32)], axis=1)

    (ov_g,) = _sc_ov_gather(idx3, ov_emb)
    comb = jnp.concatenate(
        [movie_emb, genres, rd2f,
         jnp.zeros((V, CDIM - RD_COL - 2), f32)], axis=1)
    comb_g, cw_g = _sc_small_gather(idx3, comb, coll32, wcoll_p)

    w1cb = jnp.concatenate([W1[:RD_COL], jnp.zeros((CDIM - RD_COL, 64), f32)])
    w1cw = jnp.concatenate(
        [W1[RD_COL:RD_COL + COLL_DIM], jnp.zeros((CDIM - COLL_DIM, 64), f32)])
    w1o = W1[RD_COL + COLL_DIM:RD_COL + COLL_DIM + OV_DIM]
    wts = W1[RD_COL + COLL_DIM + OV_DIM:]
    return _tc_mlp(ov_g, comb_g, cw_g, ts2, w1cb, w1cw, w1o, wts,
                   b1.reshape(1, 64), W2, b2.reshape(1, RANK))
